# baseline (speedup 1.0000x reference)
import functools

import numpy as np
import jax
import jax.numpy as jnp
from jax import lax
from jax.experimental import pallas as pl
from jax.experimental.pallas import tpu as pltpu
from jax.experimental.pallas import tpu_sc as plsc

L = 316
N = L * L
JCH = 2048
NJ = 49
NPAD = NJ * JCH
NBS = 4096
S_BLK = 1024
NS = NBS // S_BLK
_ROW_GROUPS = 4
W_BLK = 2048

_EPS = np.float32(0.1)
_ONE_MINUS_EPS = np.float32(1.0 - 0.1)
_EPS_UNIF = np.float32(0.1 * (1.0 / float(N)))
_TINY = np.float32(np.finfo(np.float32).tiny)

_KS0 = np.uint32(0)
_KS1 = np.uint32(42)
_KS2 = np.uint32(0x1BD11BDA) ^ _KS0 ^ _KS1


def _leaky(x):
    return jnp.where(x >= 0, x, np.float32(0.1) * x)


def _mlp_body(emb_ref, w1_ref, b1_ref, w2_ref, b2_ref, w3_ref, b3_ref,
              w_ref, s_ref, h_ref, acc_ref):
    t = pl.program_id(0)

    @pl.when(t == 0)
    def _():
        e = emb_ref[:]
        z1 = jax.lax.dot_general(e, w1_ref[:], (((1,), (1,)), ((), ())),
                                 preferred_element_type=jnp.float32)
        h1 = _leaky(z1 + b1_ref[:])
        z2 = jax.lax.dot_general(h1, w2_ref[:], (((1,), (1,)), ((), ())),
                                 preferred_element_type=jnp.float32)
        h2 = _leaky(z2 + b2_ref[:])
        h_ref[:] = h2
        acc_ref[0, 0] = np.float32(0.0)

    z = jax.lax.dot_general(h_ref[:], w3_ref[:], (((1,), (1,)), ((), ())),
                            preferred_element_type=jnp.float32)
    w = jax.nn.sigmoid(z + b3_ref[:])
    col = jax.lax.broadcasted_iota(jnp.int32, w.shape, 1) + t * W_BLK
    w = jnp.where(col < N, w, np.float32(0.0))
    w_ref[:] = w
    acc_ref[0, 0] += jnp.sum(w)

    @pl.when(t == pl.num_programs(0) - 1)
    def _():
        s_ref[0, 0] = acc_ref[0, 0]


def _run_mlp(emb, W1, b1, W2, b2, W3p, b3p):
    nw = NPAD // W_BLK
    w, s = pl.pallas_call(
        _mlp_body,
        grid=(nw,),
        in_specs=[
            pl.BlockSpec((1, 128), lambda t: (0, 0)),
            pl.BlockSpec((128, 128), lambda t: (0, 0)),
            pl.BlockSpec((1, 128), lambda t: (0, 0)),
            pl.BlockSpec((128, 128), lambda t: (0, 0)),
            pl.BlockSpec((1, 128), lambda t: (0, 0)),
            pl.BlockSpec((W_BLK, 128), lambda t: (t, 0)),
            pl.BlockSpec((1, W_BLK), lambda t: (0, t)),
        ],
        out_specs=[
            pl.BlockSpec((1, W_BLK), lambda t: (0, t)),
            pl.BlockSpec(memory_space=pltpu.SMEM),
        ],
        out_shape=[
            jax.ShapeDtypeStruct((1, NPAD), jnp.float32),
            jax.ShapeDtypeStruct((1, 1), jnp.float32),
        ],
        scratch_shapes=[
            pltpu.VMEM((1, 128), jnp.float32),
            pltpu.SMEM((1, 1), jnp.float32),
        ],
    )(emb, W1, b1, W2, b2, W3p, b3p)
    return w, s


def _norm_body(w_ref, s_ref, wreg_ref, logit_ref):
    t = pl.program_id(0)
    s = s_ref[0, 0]
    w = w_ref[:]
    wreg = (_ONE_MINUS_EPS * w) / s + _EPS_UNIF
    wreg_ref[:] = wreg
    col = jax.lax.broadcasted_iota(jnp.int32, w.shape, 1) + t * W_BLK
    lg = jnp.log(wreg)
    logit_ref[:] = jnp.where(col < N, lg, -jnp.inf)


def _run_norm(w, s):
    nw = NPAD // W_BLK
    return pl.pallas_call(
        _norm_body,
        grid=(nw,),
        in_specs=[
            pl.BlockSpec((1, W_BLK), lambda t: (0, t)),
            pl.BlockSpec(memory_space=pltpu.SMEM),
        ],
        out_specs=[
            pl.BlockSpec((1, W_BLK), lambda t: (0, t)),
            pl.BlockSpec((1, W_BLK), lambda t: (0, t)),
        ],
        out_shape=[
            jax.ShapeDtypeStruct((1, NPAD), jnp.float32),
            jax.ShapeDtypeStruct((1, NPAD), jnp.float32),
        ],
    )(w, s)


def _rotl(x, r):
    return (x << np.uint32(r)) | (x >> np.uint32(32 - r))


def _threefry_bits(x1):
    x0 = x1
    x1 = x0 ^ _rotl(x1, 13)
    for r in (15, 26, 6):
        x0 = x0 + x1; x1 = _rotl(x1, r); x1 = x0 ^ x1
    x0 = x0 + _KS1; x1 = x1 + (_KS2 + np.uint32(1))
    for r in (17, 29, 16, 24):
        x0 = x0 + x1; x1 = _rotl(x1, r); x1 = x0 ^ x1
    x0 = x0 + _KS2; x1 = x1 + (_KS0 + np.uint32(2))
    for r in (13, 15, 26, 6):
        x0 = x0 + x1; x1 = _rotl(x1, r); x1 = x0 ^ x1
    x0 = x0 + _KS0; x1 = x1 + (_KS1 + np.uint32(3))
    for r in (17, 29, 16, 24):
        x0 = x0 + x1; x1 = _rotl(x1, r); x1 = x0 ^ x1
    x0 = x0 + _KS1; x1 = x1 + (_KS2 + np.uint32(4))
    for r in (13, 15, 26, 6):
        x0 = x0 + x1; x1 = _rotl(x1, r); x1 = x0 ^ x1
    x0 = x0 + _KS2; x1 = x1 + (_KS0 + np.uint32(5))
    return x0 ^ x1


def _gumbel_from_bits(bits):
    fb = (bits >> np.uint32(9)) | np.uint32(0x3F800000)
    f = jax.lax.bitcast_convert_type(fb, jnp.float32) - np.float32(1.0)
    u = f + _TINY
    return -jnp.log(-jnp.log(u))


def _sample_body(logit_ref, out_ref, bv_ref, bj_ref):
    sb = pl.program_id(0)
    jb = pl.program_id(1)

    @pl.when(jb == 0)
    def _():
        bv_ref[:] = jnp.full((S_BLK, 128), -jnp.inf, jnp.float32)
        bj_ref[:] = jnp.zeros((S_BLK, 128), jnp.int32)

    srow = ((jax.lax.broadcasted_iota(jnp.int32, (1, S_BLK, 1), 1)
             + sb * S_BLK) * N)
    acc_v = bv_ref[:]
    acc_j = bj_ref[:]
    rc0 = jb * 16
    rpg = 16 // _ROW_GROUPS
    for gr in range(_ROW_GROUPS):
        jrow = (jax.lax.broadcasted_iota(jnp.int32, (rpg, 1, 128), 0) * 128
                + jax.lax.broadcasted_iota(jnp.int32, (rpg, 1, 128), 2)
                + (jb * JCH + gr * rpg * 128 + 42))
        x1 = (srow + jrow).astype(jnp.uint32)
        g = _gumbel_from_bits(_threefry_bits(x1))
        v = g + logit_ref[pl.ds(gr * rpg, rpg), :][:, None, :]
        for r in range(rpg):
            m = v[r]
            upd = m > acc_v
            acc_v = jnp.where(upd, m, acc_v)
            acc_j = jnp.where(upd, rc0 + gr * rpg + r, acc_j)
    bv_ref[:] = acc_v
    bj_ref[:] = acc_j

    @pl.when(jb == pl.num_programs(1) - 1)
    def _():
        bv = bv_ref[:]
        gv = jnp.max(bv, axis=1)
        lane = jax.lax.broadcasted_iota(jnp.int32, (S_BLK, 128), 1)
        jfull = bj_ref[:] * 128 + lane
        big = jnp.int32(np.int32(2**31 - 1))
        out_ref[:] = jnp.min(jnp.where(bv == gv[:, None], jfull, big), axis=1)


def _run_sample(logits3d):
    return pl.pallas_call(
        _sample_body,
        grid=(NS, NJ),
        in_specs=[pl.BlockSpec((16, 128), lambda s, j: (j, 0))],
        out_specs=pl.BlockSpec((S_BLK,), lambda s, j: (s,)),
        out_shape=jax.ShapeDtypeStruct((NBS,), jnp.int32),
        scratch_shapes=[
            pltpu.VMEM((S_BLK, 128), jnp.float32),
            pltpu.VMEM((S_BLK, 128), jnp.int32),
        ],
    )(logits3d)


def _run_gather_sc(samples, tab):
    info = plsc.get_sparse_core_info()
    nw = info.num_cores * info.num_subcores
    lanes = info.num_lanes
    per_w = NBS // nw
    mesh = plsc.VectorSubcoreMesh(core_axis_name="c", subcore_axis_name="s")

    @functools.partial(
        pl.kernel, mesh=mesh,
        out_type=[jax.ShapeDtypeStruct((NBS, 128), jnp.int32)] * 2,
        scratch_types=[
            pltpu.VMEM((per_w,), jnp.int32),
            pltpu.VMEM((per_w,), jnp.int32),
            pltpu.VMEM((per_w,), jnp.int32),
            pltpu.VMEM((per_w, 128), jnp.int32),
            pltpu.VMEM((per_w, 128), jnp.int32),
            pltpu.SemaphoreType.DMA,
        ],
    )
    def k(samp_hbm, tab_hbm, p1_hbm, p2_hbm,
          samp_v, l1_v, l2_v, r1_v, r2_v, sem):
        wid = lax.axis_index("s") * info.num_cores + lax.axis_index("c")
        base = wid * per_w
        pltpu.sync_copy(samp_hbm.at[pl.ds(base, per_w)], samp_v)
        lconst = jnp.full((lanes,), L, jnp.int32)
        for t in range(per_w // lanes):
            sl = pl.ds(t * lanes, lanes)
            s = samp_v[sl]
            l1 = lax.div(s, lconst)
            l1_v[sl] = l1
            l2_v[sl] = s - l1 * lconst
        pltpu.async_copy(tab_hbm.at[l1_v], r1_v, sem).wait()
        pltpu.async_copy(tab_hbm.at[l2_v], r2_v, sem).wait()
        pltpu.sync_copy(r1_v, p1_hbm.at[pl.ds(base, per_w)])
        pltpu.sync_copy(r2_v, p2_hbm.at[pl.ds(base, per_w)])

    return k(samples, tab)


def kernel(dummy_input, ops_dense, mags_dense, bs, dummy_embedding,
           W1, b1, W2, b2, W3, b3):
    del dummy_input, bs
    emb = dummy_embedding.reshape(1, 128)
    b1c = b1.reshape(1, 128)
    b2c = b2.reshape(1, 128)
    b3p = jnp.pad(b3, (0, NPAD - N)).reshape(1, NPAD)

    w, ssum = _run_mlp(emb, W1, b1c, W2, b2c, W3, b3p)
    wreg_p, logits_p = _run_norm(w, ssum)
    w_reg = wreg_p[0, :N]

    logits3d = logits_p.reshape(NJ * 16, 128)
    samples = _run_sample(logits3d)

    tab = jnp.pad(jnp.stack([ops_dense, mags_dense], axis=1),
                  ((0, 0), (0, 126)))
    p1, p2 = _run_gather_sc(samples, tab)
    return (w_reg, p1[:, 0], p1[:, 1], p2[:, 0], p2[:, 1], samples)


# speedup vs baseline: 1.2786x; 1.2786x over previous
import functools

import numpy as np
import jax
import jax.numpy as jnp
from jax import lax
from jax.experimental import pallas as pl
from jax.experimental.pallas import tpu as pltpu
from jax.experimental.pallas import tpu_sc as plsc

L = 316
N = L * L
JCH = 2048
NJ = 49
NPAD = NJ * JCH
NBS = 4096
S_BLK = 512
NS = NBS // S_BLK
_ROW_GROUPS = 4
W_BLK = 2048

_EPS = np.float32(0.1)
_ONE_MINUS_EPS = np.float32(1.0 - 0.1)
_EPS_UNIF = np.float32(0.1 * (1.0 / float(N)))
_TINY = np.float32(np.finfo(np.float32).tiny)

_KS0 = np.uint32(0)
_KS1 = np.uint32(42)
_KS2 = np.uint32(0x1BD11BDA) ^ _KS0 ^ _KS1


def _leaky(x):
    return jnp.where(x >= 0, x, np.float32(0.1) * x)


def _mlp_body(emb_ref, w1_ref, b1_ref, w2_ref, b2_ref, w3_ref, b3_ref,
              w_ref, s_ref, h_ref, acc_ref):
    t = pl.program_id(0)

    @pl.when(t == 0)
    def _():
        e = emb_ref[:]
        z1 = jax.lax.dot_general(e, w1_ref[:], (((1,), (1,)), ((), ())),
                                 preferred_element_type=jnp.float32)
        h1 = _leaky(z1 + b1_ref[:])
        z2 = jax.lax.dot_general(h1, w2_ref[:], (((1,), (1,)), ((), ())),
                                 preferred_element_type=jnp.float32)
        h2 = _leaky(z2 + b2_ref[:])
        h_ref[:] = h2
        acc_ref[0, 0] = np.float32(0.0)

    z = jax.lax.dot_general(h_ref[:], w3_ref[:], (((1,), (1,)), ((), ())),
                            preferred_element_type=jnp.float32)
    w = jax.nn.sigmoid(z + b3_ref[:])
    col = jax.lax.broadcasted_iota(jnp.int32, w.shape, 1) + t * W_BLK
    w = jnp.where(col < N, w, np.float32(0.0))
    w_ref[:] = w
    acc_ref[0, 0] += jnp.sum(w)

    @pl.when(t == pl.num_programs(0) - 1)
    def _():
        s_ref[0, 0] = acc_ref[0, 0]


def _run_mlp(emb, W1, b1, W2, b2, W3p, b3p):
    nw = NPAD // W_BLK
    w, s = pl.pallas_call(
        _mlp_body,
        grid=(nw,),
        in_specs=[
            pl.BlockSpec((1, 128), lambda t: (0, 0)),
            pl.BlockSpec((128, 128), lambda t: (0, 0)),
            pl.BlockSpec((1, 128), lambda t: (0, 0)),
            pl.BlockSpec((128, 128), lambda t: (0, 0)),
            pl.BlockSpec((1, 128), lambda t: (0, 0)),
            pl.BlockSpec((W_BLK, 128), lambda t: (t, 0)),
            pl.BlockSpec((1, W_BLK), lambda t: (0, t)),
        ],
        out_specs=[
            pl.BlockSpec((1, W_BLK), lambda t: (0, t)),
            pl.BlockSpec(memory_space=pltpu.SMEM),
        ],
        out_shape=[
            jax.ShapeDtypeStruct((1, NPAD), jnp.float32),
            jax.ShapeDtypeStruct((1, 1), jnp.float32),
        ],
        scratch_shapes=[
            pltpu.VMEM((1, 128), jnp.float32),
            pltpu.SMEM((1, 1), jnp.float32),
        ],
    )(emb, W1, b1, W2, b2, W3p, b3p)
    return w, s


def _norm_body(w_ref, s_ref, wreg_ref, logit_ref):
    t = pl.program_id(0)
    s = s_ref[0, 0]
    w = w_ref[:]
    wreg = (_ONE_MINUS_EPS * w) / s + _EPS_UNIF
    wreg_ref[:] = wreg
    col = jax.lax.broadcasted_iota(jnp.int32, w.shape, 1) + t * W_BLK
    lg = jnp.log(wreg)
    logit_ref[:] = jnp.where(col < N, lg, -jnp.inf)


def _run_norm(w, s):
    nw = NPAD // W_BLK
    return pl.pallas_call(
        _norm_body,
        grid=(nw,),
        in_specs=[
            pl.BlockSpec((1, W_BLK), lambda t: (0, t)),
            pl.BlockSpec(memory_space=pltpu.SMEM),
        ],
        out_specs=[
            pl.BlockSpec((1, W_BLK), lambda t: (0, t)),
            pl.BlockSpec((1, W_BLK), lambda t: (0, t)),
        ],
        out_shape=[
            jax.ShapeDtypeStruct((1, NPAD), jnp.float32),
            jax.ShapeDtypeStruct((1, NPAD), jnp.float32),
        ],
    )(w, s)


def _rotl(x, r):
    return (x << np.uint32(r)) | (x >> np.uint32(32 - r))


def _threefry_bits(x1):
    x0 = x1
    x1 = x0 ^ _rotl(x1, 13)
    for r in (15, 26, 6):
        x0 = x0 + x1; x1 = _rotl(x1, r); x1 = x0 ^ x1
    x0 = x0 + _KS1; x1 = x1 + (_KS2 + np.uint32(1))
    for r in (17, 29, 16, 24):
        x0 = x0 + x1; x1 = _rotl(x1, r); x1 = x0 ^ x1
    x0 = x0 + _KS2; x1 = x1 + (_KS0 + np.uint32(2))
    for r in (13, 15, 26, 6):
        x0 = x0 + x1; x1 = _rotl(x1, r); x1 = x0 ^ x1
    x0 = x0 + _KS0; x1 = x1 + (_KS1 + np.uint32(3))
    for r in (17, 29, 16, 24):
        x0 = x0 + x1; x1 = _rotl(x1, r); x1 = x0 ^ x1
    x0 = x0 + _KS1; x1 = x1 + (_KS2 + np.uint32(4))
    for r in (13, 15, 26, 6):
        x0 = x0 + x1; x1 = _rotl(x1, r); x1 = x0 ^ x1
    x0 = x0 + _KS2; x1 = x1 + (_KS0 + np.uint32(5))
    return x0 ^ x1


def _gumbel_from_bits(bits):
    fb = (bits >> np.uint32(9)) | np.uint32(0x3F800000)
    f = jax.lax.bitcast_convert_type(fb, jnp.float32) - np.float32(1.0)
    u = f + _TINY
    return -jnp.log(-jnp.log(u))


def _sample_body(logit_ref, out_ref, bv_ref, bj_ref):
    sb = pl.program_id(0)
    jb = pl.program_id(1)

    @pl.when(jb == 0)
    def _():
        bv_ref[:] = jnp.full((S_BLK, 128), -jnp.inf, jnp.float32)
        bj_ref[:] = jnp.zeros((S_BLK, 128), jnp.int32)

    srow = ((jax.lax.broadcasted_iota(jnp.int32, (1, S_BLK, 1), 1)
             + sb * S_BLK) * N)
    acc_v = bv_ref[:]
    acc_j = bj_ref[:]
    rc0 = jb * 16
    rpg = 16 // _ROW_GROUPS
    for gr in range(_ROW_GROUPS):
        jrow = (jax.lax.broadcasted_iota(jnp.int32, (rpg, 1, 128), 0) * 128
                + jax.lax.broadcasted_iota(jnp.int32, (rpg, 1, 128), 2)
                + (jb * JCH + gr * rpg * 128 + 42))
        x1 = (srow + jrow).astype(jnp.uint32)
        g = _gumbel_from_bits(_threefry_bits(x1))
        v = g + logit_ref[pl.ds(gr * rpg, rpg), :][:, None, :]
        for r in range(rpg):
            m = v[r]
            upd = m > acc_v
            acc_v = jnp.where(upd, m, acc_v)
            acc_j = jnp.where(upd, rc0 + gr * rpg + r, acc_j)
    bv_ref[:] = acc_v
    bj_ref[:] = acc_j

    @pl.when(jb == pl.num_programs(1) - 1)
    def _():
        bv = bv_ref[:]
        gv = jnp.max(bv, axis=1)
        lane = jax.lax.broadcasted_iota(jnp.int32, (S_BLK, 128), 1)
        jfull = bj_ref[:] * 128 + lane
        big = jnp.int32(np.int32(2**31 - 1))
        out_ref[:] = jnp.min(jnp.where(bv == gv[:, None], jfull, big), axis=1)


def _run_sample(logits3d):
    return pl.pallas_call(
        _sample_body,
        grid=(NS, NJ),
        in_specs=[pl.BlockSpec((16, 128), lambda s, j: (j, 0))],
        out_specs=pl.BlockSpec((S_BLK,), lambda s, j: (s,)),
        out_shape=jax.ShapeDtypeStruct((NBS,), jnp.int32),
        scratch_shapes=[
            pltpu.VMEM((S_BLK, 128), jnp.float32),
            pltpu.VMEM((S_BLK, 128), jnp.int32),
        ],
    )(logits3d)


def _run_gather_sc(samples, tab):
    info = plsc.get_sparse_core_info()
    nw = info.num_cores * info.num_subcores
    lanes = info.num_lanes
    per_w = NBS // nw
    mesh = plsc.VectorSubcoreMesh(core_axis_name="c", subcore_axis_name="s")

    @functools.partial(
        pl.kernel, mesh=mesh,
        out_type=[jax.ShapeDtypeStruct((NBS, 128), jnp.int32)] * 2,
        scratch_types=[
            pltpu.VMEM((per_w,), jnp.int32),
            pltpu.VMEM((per_w,), jnp.int32),
            pltpu.VMEM((per_w,), jnp.int32),
            pltpu.VMEM((per_w, 128), jnp.int32),
            pltpu.VMEM((per_w, 128), jnp.int32),
            pltpu.SemaphoreType.DMA,
        ],
    )
    def k(samp_hbm, tab_hbm, p1_hbm, p2_hbm,
          samp_v, l1_v, l2_v, r1_v, r2_v, sem):
        wid = lax.axis_index("s") * info.num_cores + lax.axis_index("c")
        base = wid * per_w
        pltpu.sync_copy(samp_hbm.at[pl.ds(base, per_w)], samp_v)
        lconst = jnp.full((lanes,), L, jnp.int32)
        for t in range(per_w // lanes):
            sl = pl.ds(t * lanes, lanes)
            s = samp_v[sl]
            l1 = lax.div(s, lconst)
            l1_v[sl] = l1
            l2_v[sl] = s - l1 * lconst
        pltpu.async_copy(tab_hbm.at[l1_v], r1_v, sem).wait()
        pltpu.async_copy(tab_hbm.at[l2_v], r2_v, sem).wait()
        pltpu.sync_copy(r1_v, p1_hbm.at[pl.ds(base, per_w)])
        pltpu.sync_copy(r2_v, p2_hbm.at[pl.ds(base, per_w)])

    return k(samples, tab)


def kernel(dummy_input, ops_dense, mags_dense, bs, dummy_embedding,
           W1, b1, W2, b2, W3, b3):
    del dummy_input, bs
    emb = dummy_embedding.reshape(1, 128)
    b1c = b1.reshape(1, 128)
    b2c = b2.reshape(1, 128)
    b3p = jnp.pad(b3, (0, NPAD - N)).reshape(1, NPAD)

    w, ssum = _run_mlp(emb, W1, b1c, W2, b2c, W3, b3p)
    wreg_p, logits_p = _run_norm(w, ssum)
    w_reg = wreg_p[0, :N]

    logits3d = logits_p.reshape(NJ * 16, 128)
    samples = _run_sample(logits3d)

    tab = jnp.pad(jnp.stack([ops_dense, mags_dense], axis=1),
                  ((0, 0), (0, 126)))
    p1, p2 = _run_gather_sc(samples, tab)
    return (w_reg, p1[:, 0], p1[:, 1], p2[:, 0], p2[:, 1], samples)
